# baseline (speedup 1.0000x reference)
import functools

import jax
import jax.numpy as jnp
from jax import lax
from jax.experimental import pallas as pl
from jax.experimental.pallas import tpu as pltpu
from jax.experimental.pallas import tpu_sc as plsc

NUM_EMB = 8192
DIM = 256
COMMIT = 0.25
N_TOKENS = 9216

TN = 1024
N_TILES = N_TOKENS // TN

NC, NS = 2, 16
NW = NC * NS
RPW = N_TOKENS // NW
CH = 96
NCH = RPW // CH


KC = 2048
N_KC = NUM_EMB // KC
BIG = 2 ** 30


def _argmin_body(z2_ref, e2_ref, zb_ref, emb_ref, idx_ref, minsum_ref):
    i = pl.program_id(0)
    z2 = z2_ref[...]
    tn = z2.shape[0]
    run_v = None
    run_g = None
    for j in range(N_KC):
        m = lax.dot_general(
            zb_ref[...], emb_ref[j * KC:(j + 1) * KC, :],
            dimension_numbers=(((1,), (1,)), ((), ())),
            preferred_element_type=jnp.float32)
        for t in range(KC // 128):
            g = j * (KC // 128) + t
            dgt = ((z2 + e2_ref[:, g * 128:(g + 1) * 128])
                   - 2.0 * m[:, t * 128:(t + 1) * 128])
            if run_v is None:
                run_v = dgt
                run_g = jnp.zeros((tn, 128), jnp.int32)
            else:
                mask = dgt < run_v
                run_v = jnp.where(mask, dgt, run_v)
                run_g = jnp.where(mask, jnp.full((tn, 128), g, jnp.int32),
                                  run_g)
    rowmin = jnp.min(run_v, axis=1, keepdims=True)
    lane = lax.broadcasted_iota(jnp.int32, (tn, 128), 1)
    cand = run_g * 128 + lane
    idx = jnp.min(jnp.where(run_v == rowmin, cand, BIG), axis=1)
    idx_ref[0, 0, :] = idx
    ts = jnp.sum(rowmin).reshape(1, 1)
    minsum_ref[...] = jnp.where(i == 0, ts, minsum_ref[...] + ts)


def _tc_argmin(z2, e2, flat_z, embeddings):
    return pl.pallas_call(
        _argmin_body,
        grid=(N_TILES,),
        in_specs=[
            pl.BlockSpec((TN, 1), lambda i: (i, 0)),
            pl.BlockSpec((1, NUM_EMB), lambda i: (0, 0)),
            pl.BlockSpec((TN, DIM), lambda i: (i, 0)),
            pl.BlockSpec((NUM_EMB, DIM), lambda i: (0, 0)),
        ],
        out_specs=[
            pl.BlockSpec((1, 1, TN), lambda i: (i, 0, 0)),
            pl.BlockSpec((1, 1), lambda i: (0, 0)),
        ],
        out_shape=[
            jax.ShapeDtypeStruct((N_TILES, 1, TN), jnp.int32),
            jax.ShapeDtypeStruct((1, 1), jnp.float32),
        ],
    )(z2, e2, flat_z, embeddings)


def _sc_body(emb_hbm, idx_hbm, zeros_hbm, q_hbm, counts_hbm,
             idx_v, rows_v, ones_v, counts_sh, sem):
    c = lax.axis_index("c")
    s = lax.axis_index("s")
    wid = s * NC + c
    base = wid * RPW

    @pl.when(s == 0)
    def _():
        pltpu.sync_copy(zeros_hbm, counts_sh)

    pltpu.sync_copy(idx_hbm.at[wid], idx_v)

    for t in range(CH // 16):
        ones_v[pl.ds(t * 16, 16)] = jnp.ones((16,), jnp.float32)

    descs = [
        pltpu.async_copy(emb_hbm.at[idx_v.at[j]],
                         rows_v.at[pl.ds(j * CH, CH)], sem)
        for j in range(NCH)
    ]
    for dsc in descs:
        dsc.wait()
    pltpu.sync_copy(rows_v, q_hbm.at[pl.ds(base, RPW)])

    plsc.subcore_barrier()
    for j in range(NCH):
        pltpu.sync_copy(ones_v, counts_sh.at[idx_v.at[j]], add=True)
    plsc.subcore_barrier()

    @pl.when(s == 0)
    def _():
        pltpu.sync_copy(counts_sh, counts_hbm.at[c])


@functools.lru_cache(maxsize=1)
def _sc_gather_counts():
    return pl.kernel(
        _sc_body,
        out_type=[
            jax.ShapeDtypeStruct((N_TOKENS, DIM), jnp.float32),
            jax.ShapeDtypeStruct((NC, NUM_EMB), jnp.float32),
        ],
        mesh=plsc.VectorSubcoreMesh(core_axis_name="c",
                                    subcore_axis_name="s"),
        scratch_types=[
            pltpu.VMEM((NCH, CH), jnp.int32),
            pltpu.VMEM((RPW, DIM), jnp.float32),
            pltpu.VMEM((CH,), jnp.float32),
            pltpu.VMEM_SHARED((NUM_EMB,), jnp.float32),
            pltpu.SemaphoreType.DMA,
        ],
    )


def _scalars_body(counts_ref, minsum_ref, loss_ref, perp_ref):
    cs = counts_ref[...]
    counts = cs[0:1, :] + cs[1:2, :]
    avg = counts * (1.0 / N_TOKENS)
    ent = -jnp.sum(avg * jnp.log(avg + 1e-10))
    perp_ref[...] = jnp.exp(ent).reshape(1, 1)
    loss_ref[...] = minsum_ref[...] * (COMMIT / (N_TOKENS * DIM))


def _tc_scalars(counts, minsum):
    return pl.pallas_call(
        _scalars_body,
        out_shape=[
            jax.ShapeDtypeStruct((1, 1), jnp.float32),
            jax.ShapeDtypeStruct((1, 1), jnp.float32),
        ],
    )(counts, minsum)


def kernel(z, embeddings):
    zp = jnp.transpose(z, (0, 2, 1))
    B, T, D = zp.shape
    flat_z = zp.reshape(-1, D)
    z2 = jnp.sum(flat_z ** 2, axis=1, keepdims=True)
    e2 = jnp.sum(embeddings ** 2, axis=1).reshape(1, NUM_EMB)

    idx_t, minsum = _tc_argmin(z2, e2, flat_z, embeddings)
    indices = idx_t.reshape(N_TOKENS)

    q, counts = _sc_gather_counts()(
        embeddings, indices.reshape(NW, NCH, CH),
        jnp.zeros((NUM_EMB,), jnp.float32))

    loss, perp = _tc_scalars(counts, minsum)

    quantized = jnp.transpose(q.reshape(B, T, D), (0, 2, 1))
    return (quantized, indices.reshape(B, T),
            loss.reshape(()), perp.reshape(()))


# speedup vs baseline: 1.0752x; 1.0752x over previous
import functools

import jax
import jax.numpy as jnp
from jax import lax
from jax.experimental import pallas as pl
from jax.experimental.pallas import tpu as pltpu
from jax.experimental.pallas import tpu_sc as plsc

NUM_EMB = 8192
DIM = 256
COMMIT = 0.25
N_TOKENS = 9216

TN = 512
N_TILES = N_TOKENS // TN

NC, NS = 2, 16
NW = NC * NS
RPW = N_TOKENS // NW
CH = 96
NCH = RPW // CH


KC = 2048
N_KC = NUM_EMB // KC
BIG = 2 ** 30


def _argmin_body(e2_ref, zb_ref, emb_ref, idx_ref, minsum_ref):
    i = pl.program_id(0)
    zb = zb_ref[...]
    z2 = jnp.sum(zb * zb, axis=1, keepdims=True)
    tn = z2.shape[0]
    run_v = None
    run_g = None
    for j in range(N_KC):
        m = lax.dot_general(
            zb, emb_ref[j * KC:(j + 1) * KC, :],
            dimension_numbers=(((1,), (1,)), ((), ())),
            preferred_element_type=jnp.float32)
        for t in range(KC // 128):
            g = j * (KC // 128) + t
            dgt = ((z2 + e2_ref[:, g * 128:(g + 1) * 128])
                   - 2.0 * m[:, t * 128:(t + 1) * 128])
            if run_v is None:
                run_v = dgt
                run_g = jnp.zeros((tn, 128), jnp.int32)
            else:
                mask = dgt < run_v
                run_v = jnp.where(mask, dgt, run_v)
                run_g = jnp.where(mask, jnp.full((tn, 128), g, jnp.int32),
                                  run_g)
    rowmin = jnp.min(run_v, axis=1, keepdims=True)
    lane = lax.broadcasted_iota(jnp.int32, (tn, 128), 1)
    cand = run_g * 128 + lane
    idx = jnp.min(jnp.where(run_v == rowmin, cand, BIG), axis=1)
    idx_ref[0, 0, :] = idx
    ts = jnp.sum(rowmin).reshape(1, 1)
    minsum_ref[...] = jnp.where(i == 0, ts, minsum_ref[...] + ts)


def _tc_argmin(e2, flat_z, embeddings):
    return pl.pallas_call(
        _argmin_body,
        grid=(N_TILES,),
        in_specs=[
            pl.BlockSpec((1, NUM_EMB), lambda i: (0, 0)),
            pl.BlockSpec((TN, DIM), lambda i: (i, 0)),
            pl.BlockSpec((NUM_EMB, DIM), lambda i: (0, 0)),
        ],
        out_specs=[
            pl.BlockSpec((1, 1, TN), lambda i: (i, 0, 0)),
            pl.BlockSpec((1, 1), lambda i: (0, 0)),
        ],
        out_shape=[
            jax.ShapeDtypeStruct((N_TILES, 1, TN), jnp.int32),
            jax.ShapeDtypeStruct((1, 1), jnp.float32),
        ],
    )(e2, flat_z, embeddings)


def _sc_body(emb_hbm, idx_hbm, zeros_hbm, q_hbm, counts_hbm,
             idx_v, rows_v, ones_v, counts_sh, sem):
    c = lax.axis_index("c")
    s = lax.axis_index("s")
    wid = s * NC + c
    base = wid * RPW

    @pl.when(s == 0)
    def _():
        pltpu.sync_copy(zeros_hbm, counts_sh)

    pltpu.sync_copy(idx_hbm.at[wid], idx_v)

    for t in range(CH // 16):
        ones_v[pl.ds(t * 16, 16)] = jnp.ones((16,), jnp.float32)

    descs = [
        pltpu.async_copy(emb_hbm.at[idx_v.at[j]],
                         rows_v.at[pl.ds(j * CH, CH)], sem)
        for j in range(NCH)
    ]
    for dsc in descs:
        dsc.wait()
    pltpu.sync_copy(rows_v, q_hbm.at[pl.ds(base, RPW)])

    plsc.subcore_barrier()
    for j in range(NCH):
        pltpu.sync_copy(ones_v, counts_sh.at[idx_v.at[j]], add=True)
    plsc.subcore_barrier()

    @pl.when(s == 0)
    def _():
        pltpu.sync_copy(counts_sh, counts_hbm.at[c])


@functools.lru_cache(maxsize=1)
def _sc_gather_counts():
    return pl.kernel(
        _sc_body,
        out_type=[
            jax.ShapeDtypeStruct((N_TOKENS, DIM), jnp.float32),
            jax.ShapeDtypeStruct((NC, NUM_EMB), jnp.float32),
        ],
        mesh=plsc.VectorSubcoreMesh(core_axis_name="c",
                                    subcore_axis_name="s"),
        scratch_types=[
            pltpu.VMEM((NCH, CH), jnp.int32),
            pltpu.VMEM((RPW, DIM), jnp.float32),
            pltpu.VMEM((CH,), jnp.float32),
            pltpu.VMEM_SHARED((NUM_EMB,), jnp.float32),
            pltpu.SemaphoreType.DMA,
        ],
    )


def _scalars_body(counts_ref, minsum_ref, loss_ref, perp_ref):
    cs = counts_ref[...]
    counts = cs[0:1, :] + cs[1:2, :]
    avg = counts * (1.0 / N_TOKENS)
    ent = -jnp.sum(avg * jnp.log(avg + 1e-10))
    perp_ref[...] = jnp.exp(ent).reshape(1, 1)
    loss_ref[...] = minsum_ref[...] * (COMMIT / (N_TOKENS * DIM))


def _tc_scalars(counts, minsum):
    return pl.pallas_call(
        _scalars_body,
        out_shape=[
            jax.ShapeDtypeStruct((1, 1), jnp.float32),
            jax.ShapeDtypeStruct((1, 1), jnp.float32),
        ],
    )(counts, minsum)


def kernel(z, embeddings):
    zp = jnp.transpose(z, (0, 2, 1))
    B, T, D = zp.shape
    flat_z = zp.reshape(-1, D)
    e2 = jnp.sum(embeddings ** 2, axis=1).reshape(1, NUM_EMB)

    idx_t, minsum = _tc_argmin(e2, flat_z, embeddings)
    indices = idx_t.reshape(N_TOKENS)

    q, counts = _sc_gather_counts()(
        embeddings, indices.reshape(NW, NCH, CH),
        jnp.zeros((NUM_EMB,), jnp.float32))

    loss, perp = _tc_scalars(counts, minsum)

    quantized = jnp.transpose(q.reshape(B, T, D), (0, 2, 1))
    return (quantized, indices.reshape(B, T),
            loss.reshape(()), perp.reshape(()))
